# baseline (speedup 1.0000x reference)
import numpy as np
import jax
import jax.numpy as jnp
from jax.experimental import pallas as pl
from jax.experimental.pallas import tpu as pltpu

_TB = 1024
_P = 28
_N1P = 896
_N2P = 512


def _masks1():
    bh = np.zeros((2, 3, 28, 16), np.float32)
    bw = np.zeros((2, 3, 28, 14), np.float32)
    for p in range(2):
        for d in range(3):
            for i in range(14):
                h = 2 * i + p + d - 1
                if 0 <= h < 28:
                    bh[p, d, h, i] = 1.0
            for j in range(14):
                w = 2 * j + p + d - 1
                if 0 <= w < 28:
                    bw[p, d, w, j] = 1.0
    return bh, bw


def _masks2():
    bh = np.zeros((2, 3, 16, 8), np.float32)
    bw = np.zeros((2, 3, 14, 8), np.float32)
    for p in range(2):
        for d in range(3):
            for y in range(7):
                i = 2 * y + p + d - 1
                if 0 <= i < 14:
                    bh[p, d, i, y] = 1.0
            for x in range(7):
                j = 2 * x + p + d - 1
                if 0 <= j < 14:
                    bw[p, d, j, x] = 1.0
    return bh, bw


_BH1, _BW1 = _masks1()
_BH2, _BW2 = _masks2()

_MFC = np.zeros((392, _N2P), np.float32)
for _o in range(8):
    for _y in range(7):
        for _x in range(7):
            _MFC[_o * 49 + _y * 7 + _x, _o * 64 + _y * 8 + _x] = 1.0

_MB1 = np.repeat(np.eye(4, dtype=np.float32), 224, axis=1).reshape(4, _N1P)
_MB2 = np.repeat(np.eye(8, dtype=np.float32), 64, axis=1).reshape(8, _N2P)


def _body(x_ref, a1_ref, a2_ref, a3_ref, a4_ref,
          b1_ref, b2_ref, b3_ref, b4_ref, o_ref):
    f32 = jnp.float32
    xb = x_ref[...].astype(jnp.bfloat16)

    c = jnp.dot(xb, a1_ref[...], preferred_element_type=f32)
    h1 = jnp.maximum(
        jnp.maximum(c[:, 0 * _N1P:1 * _N1P], c[:, 1 * _N1P:2 * _N1P]),
        jnp.maximum(c[:, 2 * _N1P:3 * _N1P], c[:, 3 * _N1P:4 * _N1P]))
    h1 = jnp.maximum(h1 + b1_ref[...], 0.0)
    h1 = h1.astype(jnp.bfloat16)

    c = jnp.dot(h1, a2_ref[...], preferred_element_type=f32)
    h2 = jnp.maximum(
        jnp.maximum(c[:, 0 * _N2P:1 * _N2P], c[:, 1 * _N2P:2 * _N2P]),
        jnp.maximum(c[:, 2 * _N2P:3 * _N2P], c[:, 3 * _N2P:4 * _N2P]))
    h2 = jnp.maximum(h2 + b2_ref[...], 0.0)
    h2 = h2.astype(jnp.bfloat16)

    f = jnp.dot(h2, a3_ref[...], preferred_element_type=f32) + b3_ref[...]
    f = jnp.maximum(f, 0.0).astype(jnp.bfloat16)
    o_ref[...] = jnp.dot(f, a4_ref[...], preferred_element_type=f32) \
        + b4_ref[...]


def kernel(w1, b1, w2, b2, fc1_w, fc1_b, fc2_w, fc2_b, x):
    B = x.shape[0]
    nc = fc2_w.shape[0]
    b_pad = -(-B // _TB) * _TB
    x2 = x.reshape(B, _P * _P).astype(jnp.float32)
    if b_pad != B:
        x2 = jnp.pad(x2, ((0, b_pad - B), (0, 0)))

    f32 = jnp.float32
    w1r = w1.reshape(4, 3, 3).astype(f32)
    w2r = w2.astype(f32)
    a1 = jnp.einsum('cde,pdHi,qeWj->HWpqcij', w1r, _BH1, _BW1) \
        .reshape(_P * _P, 4 * _N1P).astype(jnp.bfloat16)
    a2 = jnp.einsum('ocde,pdIy,qeJx->cIJpqoyx', w2r, _BH2, _BW2) \
        .reshape(_N1P, 4 * _N2P).astype(jnp.bfloat16)
    a3 = jnp.einsum('ck,kl->lc', fc1_w.astype(f32), _MFC).astype(jnp.bfloat16)
    a4 = fc2_w.astype(f32).T.astype(jnp.bfloat16)
    b1l = (b1.astype(f32) @ _MB1).reshape(1, _N1P)
    b2l = (b2.astype(f32) @ _MB2).reshape(1, _N2P)
    b3l = fc1_b.astype(f32).reshape(1, 32)
    b4l = fc2_b.astype(f32).reshape(1, nc)

    out = pl.pallas_call(
        _body,
        out_shape=jax.ShapeDtypeStruct((b_pad, nc), jnp.float32),
        grid=(b_pad // _TB,),
        in_specs=[
            pl.BlockSpec((_TB, _P * _P), lambda i: (i, 0)),
            pl.BlockSpec((_P * _P, 4 * _N1P), lambda i: (0, 0)),
            pl.BlockSpec((_N1P, 4 * _N2P), lambda i: (0, 0)),
            pl.BlockSpec((_N2P, 32), lambda i: (0, 0)),
            pl.BlockSpec((32, nc), lambda i: (0, 0)),
            pl.BlockSpec((1, _N1P), lambda i: (0, 0)),
            pl.BlockSpec((1, _N2P), lambda i: (0, 0)),
            pl.BlockSpec((1, 32), lambda i: (0, 0)),
            pl.BlockSpec((1, nc), lambda i: (0, 0)),
        ],
        out_specs=pl.BlockSpec((_TB, nc), lambda i: (i, 0)),
        compiler_params=pltpu.CompilerParams(
            dimension_semantics=("parallel",),
            vmem_limit_bytes=64 * 1024 * 1024,
        ),
    )(x2, a1, a2, a3, a4, b1l, b2l, b3l, b4l)
    return out[:B]
